# baseline (speedup 1.0000x reference)
import jax
import jax.numpy as jnp
from jax.experimental import pallas as pl
from jax.experimental.pallas import tpu as pltpu

_B, _N, _NB, _D, _E, _K, _F, _T, _H = 1024, 26, 16, 64, 8, 2, 512, 2, 256
_DM = _N * _D
_NNB = _N * _NB
_BM1 = 512
_NB1 = _B // _BM1


def _embed_route_body(xr_ref, left_ref, invw_ref, wbd_ref, bemb_ref,
                      wr_ref, tok_ref, comb_ref):
    enc = jnp.clip((xr_ref[...] - left_ref[...]) * invw_ref[...], 0.0, 1.0)
    tok = jnp.dot(enc, wbd_ref[...], preferred_element_type=jnp.float32)
    tok = tok + bemb_ref[...]
    logits = jnp.dot(tok, wr_ref[...], preferred_element_type=jnp.float32)
    lane = jax.lax.broadcasted_iota(jnp.int32, (_BM1, _E), 1)
    m1 = jnp.max(logits, axis=1, keepdims=True)
    i1 = jnp.min(jnp.where(logits == m1, lane, _E), axis=1, keepdims=True)
    l2 = jnp.where(lane == i1, -jnp.inf, logits)
    m2 = jnp.max(l2, axis=1, keepdims=True)
    i2 = jnp.min(jnp.where(l2 == m2, lane, _E), axis=1, keepdims=True)
    t = jnp.exp(m2 - m1)
    g1 = 1.0 / (1.0 + t)
    g2 = t / (1.0 + t)
    comb = jnp.where(lane == i1, g1, 0.0) + jnp.where(lane == i2, g2, 0.0)
    tok_ref[...] = tok.astype(jnp.bfloat16)
    comb_ref[...] = comb


def _moe_body(tok_ref, comb_ref, w1_ref, b1_ref, w2_ref, b2_ref,
              wt1_ref, bt1_ref, wt2_ref, bt2_ref, out_ref,
              h_ref, w1bf_ref, w2bf_ref):
    e = pl.program_id(0)

    w1bf_ref[...] = w1_ref[0].astype(jnp.bfloat16)
    w2bf_ref[pl.ds(e * _F, _F), :] = w2_ref[0].astype(jnp.bfloat16)

    tok = tok_ref[...]
    h = jnp.dot(tok, w1bf_ref[...], preferred_element_type=jnp.float32)
    h = jnp.maximum(h + b1_ref[0], 0.0)
    lane = jax.lax.broadcasted_iota(jnp.int32, (_B, _E), 1)
    ge = jnp.sum(jnp.where(lane == e, comb_ref[...], 0.0), axis=1,
                 keepdims=True)
    h_ref[:, pl.ds(e * _F, _F)] = (ge * h).astype(jnp.bfloat16)

    @pl.when(e == _E - 1)
    def _():
        moe = jnp.dot(h_ref[...], w2bf_ref[...],
                      preferred_element_type=jnp.float32)
        moe = moe + jnp.dot(comb_ref[...], b2_ref[...],
                            preferred_element_type=jnp.float32)
        moe = moe.astype(jnp.bfloat16)
        for t in range(_T):
            wt1 = wt1_ref[t].astype(jnp.bfloat16)
            th = jnp.dot(moe, wt1, preferred_element_type=jnp.float32)
            th = jnp.maximum(th + bt1_ref[t:t + 1, :], 0.0)
            th = th.astype(jnp.bfloat16)
            wt2 = wt2_ref[t].astype(jnp.bfloat16)
            o = jnp.dot(th, wt2, preferred_element_type=jnp.float32)
            out_ref[:, t:t + 1] = o + bt2_ref[t:t + 1, :]


def kernel(x, edges, W_emb, b_emb, W_router, W1, b1, W2, b2,
           Wt1, bt1, Wt2, bt2):
    xr = jnp.repeat(x, _NB, axis=1)
    left = edges[:, :-1].reshape(1, _NNB)
    invw = 1.0 / (edges[:, 1:] - edges[:, :-1]).reshape(1, _NNB)
    wbd = jnp.zeros((_NNB, _DM), jnp.float32)
    for n in range(_N):
        wbd = wbd.at[n * _NB:(n + 1) * _NB, n * _D:(n + 1) * _D].set(W_emb[n])
    bemb = b_emb.reshape(1, _DM)

    tok, comb = pl.pallas_call(
        _embed_route_body,
        grid=(_NB1,),
        in_specs=[
            pl.BlockSpec((_BM1, _NNB), lambda b: (b, 0)),
            pl.BlockSpec((1, _NNB), lambda b: (0, 0)),
            pl.BlockSpec((1, _NNB), lambda b: (0, 0)),
            pl.BlockSpec((_NNB, _DM), lambda b: (0, 0)),
            pl.BlockSpec((1, _DM), lambda b: (0, 0)),
            pl.BlockSpec((_DM, _E), lambda b: (0, 0)),
        ],
        out_specs=[
            pl.BlockSpec((_BM1, _DM), lambda b: (b, 0)),
            pl.BlockSpec((_BM1, _E), lambda b: (b, 0)),
        ],
        out_shape=[
            jax.ShapeDtypeStruct((_B, _DM), jnp.bfloat16),
            jax.ShapeDtypeStruct((_B, _E), jnp.float32),
        ],
    )(xr, left, invw, wbd, bemb, W_router)

    out = pl.pallas_call(
        _moe_body,
        grid=(_E,),
        in_specs=[
            pl.BlockSpec((_B, _DM), lambda e: (0, 0)),
            pl.BlockSpec((_B, _E), lambda e: (0, 0)),
            pl.BlockSpec((1, _DM, _F), lambda e: (e, 0, 0)),
            pl.BlockSpec((1, 1, _F), lambda e: (e, 0, 0)),
            pl.BlockSpec((1, _F, _DM), lambda e: (e, 0, 0)),
            pl.BlockSpec((_E, _DM), lambda e: (0, 0)),
            pl.BlockSpec((_T, _DM, _H), lambda e: (0, 0, 0)),
            pl.BlockSpec((_T, _H), lambda e: (0, 0)),
            pl.BlockSpec((_T, _H, 1), lambda e: (0, 0, 0)),
            pl.BlockSpec((_T, 1), lambda e: (0, 0)),
        ],
        out_specs=pl.BlockSpec((_B, _T), lambda e: (0, 0)),
        out_shape=jax.ShapeDtypeStruct((_B, _T), jnp.float32),
        scratch_shapes=[
            pltpu.VMEM((_B, _E * _F), jnp.bfloat16),
            pltpu.VMEM((_DM, _F), jnp.bfloat16),
            pltpu.VMEM((_E * _F, _DM), jnp.bfloat16),
        ],
        compiler_params=pltpu.CompilerParams(
            dimension_semantics=("arbitrary",)),
    )(tok, comb, W1, b1.reshape(_E, 1, _F), W2, b2,
      Wt1, bt1, Wt2, bt2)
    return out


# speedup vs baseline: 1.2506x; 1.2506x over previous
import jax
import jax.numpy as jnp
from jax.experimental import pallas as pl
from jax.experimental.pallas import tpu as pltpu

_B, _N, _NB, _D, _E, _K, _F, _T, _H = 1024, 26, 16, 64, 8, 2, 512, 2, 256
_DM = _N * _D
_NNB = _N * _NB
_BM1 = 512
_NB1 = _B // _BM1


def _embed_route_body(x_ref, rep_ref, left_ref, invw_ref, wembf_ref, tile_ref,
                      mask_ref, bemb_ref, wr_ref, tok_ref, comb_ref, wbd_ref):
    b = pl.program_id(0)

    @pl.when(b == 0)
    def _():
        wbd_ref[...] = jnp.dot(wembf_ref[...], tile_ref[...],
                               preferred_element_type=jnp.float32) * mask_ref[...]

    xr = jnp.dot(x_ref[...], rep_ref[...], preferred_element_type=jnp.float32)
    enc = jnp.clip((xr - left_ref[...]) * invw_ref[...], 0.0, 1.0)
    tok = jnp.dot(enc, wbd_ref[...], preferred_element_type=jnp.float32)
    tok = tok + bemb_ref[...]
    logits = jnp.dot(tok, wr_ref[...], preferred_element_type=jnp.float32)
    lane = jax.lax.broadcasted_iota(jnp.int32, (_BM1, _E), 1)
    m1 = jnp.max(logits, axis=1, keepdims=True)
    i1 = jnp.min(jnp.where(logits == m1, lane, _E), axis=1, keepdims=True)
    l2 = jnp.where(lane == i1, -jnp.inf, logits)
    m2 = jnp.max(l2, axis=1, keepdims=True)
    i2 = jnp.min(jnp.where(l2 == m2, lane, _E), axis=1, keepdims=True)
    t = jnp.exp(m2 - m1)
    g1 = 1.0 / (1.0 + t)
    g2 = t / (1.0 + t)
    comb = jnp.where(lane == i1, g1, 0.0) + jnp.where(lane == i2, g2, 0.0)
    tok_ref[...] = tok.astype(jnp.bfloat16)
    comb_ref[...] = comb


def _moe_body(tok_ref, comb_ref, w1_ref, b1_ref, w2_ref, b2_ref,
              wt1_ref, bt1_ref, wt2_ref, bt2_ref, out_ref,
              h_ref, w1bf_ref, w2bf_ref):
    e = pl.program_id(0)

    w1bf_ref[...] = w1_ref[0].astype(jnp.bfloat16)
    w2bf_ref[pl.ds(e * _F, _F), :] = w2_ref[0].astype(jnp.bfloat16)

    tok = tok_ref[...]
    h = jnp.dot(tok, w1bf_ref[...], preferred_element_type=jnp.float32)
    h = jnp.maximum(h + b1_ref[0], 0.0)
    lane = jax.lax.broadcasted_iota(jnp.int32, (_B, _E), 1)
    ge = jnp.sum(jnp.where(lane == e, comb_ref[...], 0.0), axis=1,
                 keepdims=True)
    h_ref[:, pl.ds(e * _F, _F)] = (ge * h).astype(jnp.bfloat16)

    @pl.when(e == _E - 1)
    def _():
        moe = jnp.dot(h_ref[...], w2bf_ref[...],
                      preferred_element_type=jnp.float32)
        moe = moe + jnp.dot(comb_ref[...], b2_ref[...],
                            preferred_element_type=jnp.float32)
        moe = moe.astype(jnp.bfloat16)
        for t in range(_T):
            wt1 = wt1_ref[t].astype(jnp.bfloat16)
            th = jnp.dot(moe, wt1, preferred_element_type=jnp.float32)
            th = jnp.maximum(th + bt1_ref[t:t + 1, :], 0.0)
            th = th.astype(jnp.bfloat16)
            wt2 = wt2_ref[t].astype(jnp.bfloat16)
            o = jnp.dot(th, wt2, preferred_element_type=jnp.float32)
            out_ref[:, t:t + 1] = o + bt2_ref[t:t + 1, :]


def kernel(x, edges, W_emb, b_emb, W_router, W1, b1, W2, b2,
           Wt1, bt1, Wt2, bt2):
    left = edges[:, :-1].reshape(1, _NNB)
    invw = 1.0 / (edges[:, 1:] - edges[:, :-1]).reshape(1, _NNB)
    bemb = b_emb.reshape(1, _DM)
    rep = jnp.repeat(jnp.eye(_N, dtype=jnp.float32), _NB, axis=1)
    tile = jnp.tile(jnp.eye(_D, dtype=jnp.float32), (1, _N))
    rows = jnp.arange(_NNB)[:, None] // _NB
    cols = jnp.arange(_DM)[None, :] // _D
    mask = (rows == cols).astype(jnp.float32)
    wembf = W_emb.reshape(_NNB, _D)

    tok, comb = pl.pallas_call(
        _embed_route_body,
        grid=(_NB1,),
        in_specs=[
            pl.BlockSpec((_BM1, _N), lambda b: (b, 0)),
            pl.BlockSpec((_N, _NNB), lambda b: (0, 0)),
            pl.BlockSpec((1, _NNB), lambda b: (0, 0)),
            pl.BlockSpec((1, _NNB), lambda b: (0, 0)),
            pl.BlockSpec((_NNB, _D), lambda b: (0, 0)),
            pl.BlockSpec((_D, _DM), lambda b: (0, 0)),
            pl.BlockSpec((_NNB, _DM), lambda b: (0, 0)),
            pl.BlockSpec((1, _DM), lambda b: (0, 0)),
            pl.BlockSpec((_DM, _E), lambda b: (0, 0)),
        ],
        out_specs=[
            pl.BlockSpec((_BM1, _DM), lambda b: (b, 0)),
            pl.BlockSpec((_BM1, _E), lambda b: (b, 0)),
        ],
        out_shape=[
            jax.ShapeDtypeStruct((_B, _DM), jnp.bfloat16),
            jax.ShapeDtypeStruct((_B, _E), jnp.float32),
        ],
        scratch_shapes=[pltpu.VMEM((_NNB, _DM), jnp.float32)],
    )(x, rep, left, invw, wembf, tile, mask, bemb, W_router)

    out = pl.pallas_call(
        _moe_body,
        grid=(_E,),
        in_specs=[
            pl.BlockSpec((_B, _DM), lambda e: (0, 0)),
            pl.BlockSpec((_B, _E), lambda e: (0, 0)),
            pl.BlockSpec((1, _DM, _F), lambda e: (e, 0, 0)),
            pl.BlockSpec((1, 1, _F), lambda e: (e, 0, 0)),
            pl.BlockSpec((1, _F, _DM), lambda e: (e, 0, 0)),
            pl.BlockSpec((_E, _DM), lambda e: (0, 0)),
            pl.BlockSpec((_T, _DM, _H), lambda e: (0, 0, 0)),
            pl.BlockSpec((_T, _H), lambda e: (0, 0)),
            pl.BlockSpec((_T, _H, 1), lambda e: (0, 0, 0)),
            pl.BlockSpec((_T, 1), lambda e: (0, 0)),
        ],
        out_specs=pl.BlockSpec((_B, _T), lambda e: (0, 0)),
        out_shape=jax.ShapeDtypeStruct((_B, _T), jnp.float32),
        scratch_shapes=[
            pltpu.VMEM((_B, _E * _F), jnp.bfloat16),
            pltpu.VMEM((_DM, _F), jnp.bfloat16),
            pltpu.VMEM((_E * _F, _DM), jnp.bfloat16),
        ],
        compiler_params=pltpu.CompilerParams(
            dimension_semantics=("arbitrary",)),
    )(tok, comb, W1, b1.reshape(_E, 1, _F), W2, b2,
      Wt1, bt1, Wt2, bt2)
    return out
